# baseline (speedup 1.0000x reference)
import functools

import jax
import jax.numpy as jnp
from jax import lax
from jax.experimental import pallas as pl
from jax.experimental.pallas import tpu as pltpu
from jax.experimental.pallas import tpu_sc as plsc

VOCAB = 1000000
EMB = 64
B = 4096
L = 200

NC, NS, LANES = 2, 16, 16
NW = NC * NS

TCOLS = (VOCAB + 127) // 128
A_ITERS = (TCOLS + NW - 1) // NW
W_ROWS = TCOLS * 64
BLK = B // NW

_MESH = plsc.VectorSubcoreMesh(core_axis_name="c", subcore_axis_name="s")
_PARAMS = pltpu.CompilerParams(use_tc_tiling_on_sc=True,
                               needs_layout_passes=False)


def _wid():
    return lax.axis_index("s") * NC + lax.axis_index("c")


@functools.partial(
    pl.kernel,
    out_type=jax.ShapeDtypeStruct((W_ROWS, 128), jnp.float32),
    mesh=_MESH,
    compiler_params=_PARAMS,
    scratch_types=[
        pltpu.VMEM((2, EMB, 128), jnp.float32),
        pltpu.VMEM((2, EMB, 128), jnp.float32),
        pltpu.SemaphoreType.DMA,
        pltpu.SemaphoreType.DMA,
        pltpu.SemaphoreType.DMA,
        pltpu.SemaphoreType.DMA,
    ],
)
def _pack_kernel(tt_hbm, w_hbm, stage_v, pk_v, r0, r1, w0, w1):
    rsem = [r0, r1]
    wsem = [w0, w1]
    wid = _wid()

    iota = lax.iota(jnp.int32, LANES)
    rows_q = [(iota + 16 * q) % EMB for q in range(8)]
    colq = [jnp.full((LANES,), q // 4, jnp.int32) for q in range(8)]

    def fire_read(s, i):
        tc = wid + NW * i

        @pl.when(tc < TCOLS)
        def _():
            pltpu.async_copy(
                tt_hbm.at[:, pl.ds(tc * 128, 128)], stage_v.at[s], rsem[s])

    def drain(sem, ref):
        pltpu.make_async_copy(w_hbm.at[pl.ds(0, EMB)], ref, sem).wait()

    fire_read(0, 0)

    def pair(p, carry):
        for s in range(2):
            i = 2 * p + s
            fire_read(1 - s, i + 1)
            tc = wid + NW * i

            @pl.when(tc < TCOLS)
            def _():
                drain(rsem[s], stage_v.at[s])

                @pl.when(i >= 2)
                def _():
                    drain(wsem[s], pk_v.at[s])

                def row(j, c):
                    for q in range(8):
                        v = plsc.load_gather(
                            stage_v.at[s], [rows_q[q], colq[q] + 2 * j])
                        pk_v[s, j, pl.ds(16 * q, 16)] = v
                    return c

                lax.fori_loop(0, EMB, row, 0)
                pltpu.async_copy(
                    pk_v.at[s], w_hbm.at[pl.ds(tc * EMB, EMB)], wsem[s])

        return carry

    lax.fori_loop(0, (A_ITERS + 1) // 2, pair, 0)
    for i in (A_ITERS - 2, A_ITERS - 1):
        @pl.when(wid + NW * i < TCOLS)
        def _():
            drain(wsem[i % 2], pk_v.at[i % 2])


@functools.partial(
    pl.kernel,
    out_type=jax.ShapeDtypeStruct((L, EMB, B), jnp.float32),
    mesh=_MESH,
    compiler_params=_PARAMS,
    scratch_types=[
        pltpu.VMEM((L, 128), jnp.int32),
        pltpu.VMEM((L, 128), jnp.int32),
        pltpu.VMEM((2, 128, 128), jnp.float32),
        pltpu.VMEM((2, EMB, 128), jnp.float32),
        pltpu.SemaphoreType.DMA,
        pltpu.SemaphoreType.DMA,
        pltpu.SemaphoreType.DMA,
        pltpu.SemaphoreType.DMA,
    ],
)
def _gather_kernel(st_hbm, w_hbm, out_hbm, idx_v, off_v, g_v, ob_v,
                   g0, g1, w0, w1):
    gsem = [g0, g1]
    wsem = [w0, w1]
    wid = _wid()
    b0 = wid * BLK

    pltpu.sync_copy(st_hbm.at[:, pl.ds(b0, BLK)], idx_v)

    def mkoff(l, c):
        for k in range(8):
            v = idx_v[l, pl.ds(16 * k, 16)]
            off_v[l, pl.ds(16 * k, 16)] = lax.shift_right_logical(v, 1)
        return c

    lax.fori_loop(0, L, mkoff, 0)

    iota = lax.iota(jnp.int32, LANES)

    def fire_gather(s, l):
        @pl.when(l < L)
        def _():
            pltpu.async_copy(w_hbm.at[off_v.at[l]], g_v.at[s], gsem[s])

    def drain(sem, ref, n):
        pltpu.make_async_copy(w_hbm.at[pl.ds(0, n)], ref, sem).wait()

    fire_gather(0, 0)

    def pair(p, carry):
        for s in range(2):
            l = 2 * p + s
            fire_gather(1 - s, l + 1)
            drain(gsem[s], g_v.at[s], 128)

            @pl.when(l >= 2)
            def _():
                drain(wsem[s], ob_v.at[s], EMB)

            for k in range(8):
                par = lax.bitwise_and(idx_v[l, pl.ds(16 * k, 16)],
                                      jnp.int32(1)) * EMB
                rows_k = iota + 16 * k

                def erow(e, c):
                    v = plsc.load_gather(g_v.at[s], [rows_k, par + e])
                    ob_v[s, e, pl.ds(16 * k, 16)] = v
                    return c

                lax.fori_loop(0, EMB, erow, 0)

            pltpu.async_copy(
                ob_v.at[s], out_hbm.at[l, :, pl.ds(b0, BLK)], wsem[s])

        return carry

    lax.fori_loop(0, L // 2, pair, 0)
    drain(wsem[(L - 2) % 2], ob_v.at[(L - 2) % 2], EMB)
    drain(wsem[(L - 1) % 2], ob_v.at[(L - 1) % 2], EMB)


def kernel(src, mask, emb_table):
    del mask
    tt = jnp.swapaxes(emb_table, 0, 1)
    st = jnp.swapaxes(src, 0, 1)
    w = _pack_kernel(tt)
    out_t = _gather_kernel(st, w)
    return jnp.transpose(out_t, (2, 0, 1))


# speedup vs baseline: 1.5491x; 1.5491x over previous
import functools

import jax
import jax.numpy as jnp
from jax import lax
from jax.experimental import pallas as pl
from jax.experimental.pallas import tpu as pltpu
from jax.experimental.pallas import tpu_sc as plsc

VOCAB = 1000000
EMB = 64
B = 4096
L = 200

NC, NS, LANES = 2, 16, 16
NW = NC * NS
BLK = B // NW

_MESH = plsc.VectorSubcoreMesh(core_axis_name="c", subcore_axis_name="s")
_PARAMS = pltpu.CompilerParams(use_tc_tiling_on_sc=True,
                               needs_layout_passes=False)


@functools.partial(
    pl.kernel,
    out_type=jax.ShapeDtypeStruct((L, EMB, B), jnp.float32),
    mesh=_MESH,
    compiler_params=_PARAMS,
    scratch_types=[
        pltpu.VMEM((L, 128), jnp.int32),
        pltpu.VMEM((2, 128, 128), jnp.float32),
        pltpu.VMEM((2, EMB, 128), jnp.float32),
        pltpu.SemaphoreType.DMA,
        pltpu.SemaphoreType.DMA,
        pltpu.SemaphoreType.DMA,
        pltpu.SemaphoreType.DMA,
    ],
)
def _gather_kernel(st_hbm, w_hbm, out_hbm, idx_v, g_v, ob_v, g0, g1, w0, w1):
    gsem = [g0, g1]
    wsem = [w0, w1]
    wid = lax.axis_index("s") * NC + lax.axis_index("c")
    b0 = wid * BLK

    pltpu.sync_copy(st_hbm.at[:, pl.ds(b0, BLK)], idx_v)

    iota = lax.iota(jnp.int32, LANES)
    rows_k = [iota + 16 * k for k in range(8)]

    def fire_gather(s, l):
        @pl.when(l < L)
        def _():
            pltpu.async_copy(w_hbm.at[idx_v.at[l]], g_v.at[s], gsem[s])

    def drain(sem, ref, n):
        pltpu.make_async_copy(w_hbm.at[pl.ds(0, n)], ref, sem).wait()

    fire_gather(0, 0)

    def pair(p, carry):
        for s in range(2):
            l = 2 * p + s
            fire_gather(1 - s, l + 1)
            drain(gsem[s], g_v.at[s], 128)

            @pl.when(l >= 2)
            def _():
                drain(wsem[s], ob_v.at[s], EMB)

            def erow(e, c):
                for k in range(8):
                    v = plsc.load_gather(g_v.at[s], [rows_k[k], iota * 0 + e])
                    ob_v[s, e, pl.ds(16 * k, 16)] = v
                return c

            lax.fori_loop(0, EMB, erow, 0)

            pltpu.async_copy(
                ob_v.at[s], out_hbm.at[l, :, pl.ds(b0, BLK)], wsem[s])

        return carry

    lax.fori_loop(0, L // 2, pair, 0)
    drain(wsem[0], ob_v.at[0], EMB)
    drain(wsem[1], ob_v.at[1], EMB)


def kernel(src, mask, emb_table):
    del mask
    w = jnp.pad(emb_table, ((0, 0), (0, 128 - EMB)))
    st = jnp.swapaxes(src, 0, 1)
    out_t = _gather_kernel(st, w)
    return jnp.transpose(out_t, (2, 0, 1))


# speedup vs baseline: 1.5517x; 1.0016x over previous
import functools

import jax
import jax.numpy as jnp
from jax import lax
from jax.experimental import pallas as pl
from jax.experimental.pallas import tpu as pltpu
from jax.experimental.pallas import tpu_sc as plsc

VOCAB = 1000000
EMB = 64
B = 4096
L = 200

NC, NS, LANES = 2, 16, 16
NW = NC * NS
BLK = B // NW

_MESH = plsc.VectorSubcoreMesh(core_axis_name="c", subcore_axis_name="s")
_PARAMS = pltpu.CompilerParams(use_tc_tiling_on_sc=True,
                               needs_layout_passes=False)


@functools.partial(
    pl.kernel,
    out_type=jax.ShapeDtypeStruct((L, EMB, B), jnp.float32),
    mesh=_MESH,
    compiler_params=_PARAMS,
    scratch_types=[
        pltpu.VMEM((L, 128), jnp.int32),
        pltpu.VMEM((2, 128, 128), jnp.float32),
        pltpu.VMEM((2, EMB, 128), jnp.float32),
        pltpu.SemaphoreType.DMA,
        pltpu.SemaphoreType.DMA,
        pltpu.SemaphoreType.DMA,
        pltpu.SemaphoreType.DMA,
    ],
)
def _gather_kernel(st_hbm, w_hbm, out_hbm, idx_v, g_v, ob_v, g0, g1, w0, w1):
    gsem = [g0, g1]
    wsem = [w0, w1]
    wid = lax.axis_index("s") * NC + lax.axis_index("c")
    b0 = wid * BLK

    pltpu.sync_copy(st_hbm.at[:, pl.ds(b0, BLK)], idx_v)

    iota = lax.iota(jnp.int32, LANES)
    rows_k = [iota + 16 * k for k in range(8)]

    def fire_gather(s, l):
        @pl.when(l < L)
        def _():
            pltpu.async_copy(w_hbm.at[idx_v.at[l]], g_v.at[s], gsem[s])

    def drain(sem, ref, n):
        pltpu.make_async_copy(w_hbm.at[pl.ds(0, n)], ref, sem).wait()

    fire_gather(0, 0)

    def pair(p, carry):
        for s in range(2):
            l = 2 * p + s
            fire_gather(1 - s, l + 1)
            drain(gsem[s], g_v.at[s], 128)

            @pl.when(l >= 2)
            def _():
                drain(wsem[s], ob_v.at[s], EMB)

            def erow(e4, c):
                e0 = e4 * 4
                for d in range(4):
                    col = jnp.full((LANES,), 0, jnp.int32) + (e0 + d)
                    for k in range(8):
                        v = plsc.load_gather(g_v.at[s], [rows_k[k], col])
                        ob_v[s, e0 + d, pl.ds(16 * k, 16)] = v
                return c

            lax.fori_loop(0, EMB // 4, erow, 0)

            pltpu.async_copy(
                ob_v.at[s], out_hbm.at[l, :, pl.ds(b0, BLK)], wsem[s])

        return carry

    lax.fori_loop(0, L // 2, pair, 0)
    drain(wsem[0], ob_v.at[0], EMB)
    drain(wsem[1], ob_v.at[1], EMB)


def kernel(src, mask, emb_table):
    del mask
    w = jnp.pad(emb_table, ((0, 0), (0, 128 - EMB)))
    st = jnp.swapaxes(src, 0, 1)
    out_t = _gather_kernel(st, w)
    return jnp.transpose(out_t, (2, 0, 1))
